# baseline (speedup 1.0000x reference)
import jax
import jax.numpy as jnp
from jax.experimental import pallas as pl
from jax.experimental.pallas import tpu as pltpu

_TOP_K = 8


def _topk_softmax_chunk(s, iota, ef):
    vals = []
    idxs = []
    for k in range(_TOP_K):
        m = jnp.max(s, axis=1, keepdims=True)
        eq = s == m
        hit = jnp.where(eq, iota, ef)
        idx = jnp.min(hit, axis=1, keepdims=True)
        vals.append(m)
        idxs.append(idx)
        if k + 1 < _TOP_K:
            s = jnp.where(eq, -jnp.inf, s)
    v = jnp.concatenate(vals, axis=1)
    ix = jnp.concatenate(idxs, axis=1)
    ex = jnp.exp(v - v[:, 0:1])
    return ex / jnp.sum(ex, axis=1, keepdims=True), ix.astype(jnp.int32)


def _dot_wt(x, w):
    return jax.lax.dot_general(
        x, w, (((1,), (1,)), ((), ())), preferred_element_type=jnp.float32)


def _router_body(x_ref, w_ref, probs_ref, idx_ref):
    bt = x_ref.shape[1]
    e = w_ref.shape[0]
    h = bt // 2
    w = w_ref[...]
    rc = 64
    iota = jax.lax.broadcasted_iota(jnp.int32, (rc, e), 1).astype(jnp.float32)
    ef = float(e)
    s1 = _dot_wt(x_ref[0, 0:h, :], w)
    nchunks = h // rc
    s2_pieces = []
    out1 = []
    for c in range(nchunks):
        lo = h + c * rc
        s2_pieces.append(_dot_wt(x_ref[0, lo:lo + rc, :], w))
        out1.append(_topk_softmax_chunk(s1[c * rc:(c + 1) * rc, :], iota, ef))
    for c in range(nchunks):
        p, ix = out1[c]
        probs_ref[0, c * rc:(c + 1) * rc, :] = p
        idx_ref[0, c * rc:(c + 1) * rc, :] = ix
        p2, ix2 = _topk_softmax_chunk(s2_pieces[c], iota, ef)
        lo = h + c * rc
        probs_ref[0, lo:lo + rc, :] = p2
        idx_ref[0, lo:lo + rc, :] = ix2


def kernel(x, W_gate):
    b, s, d = x.shape
    e = W_gate.shape[0]
    t = b * s
    bt = min(512, s)
    grid = (t // bt,)
    spb = s // bt
    probs, idx = pl.pallas_call(
        _router_body,
        grid=grid,
        in_specs=[
            pl.BlockSpec((1, bt, d), lambda i: (i // spb, i % spb, 0)),
            pl.BlockSpec((e, d), lambda i: (0, 0)),
        ],
        out_specs=[
            pl.BlockSpec((1, bt, _TOP_K), lambda i: (i // spb, i % spb, 0)),
            pl.BlockSpec((1, bt, _TOP_K), lambda i: (i // spb, i % spb, 0)),
        ],
        out_shape=[
            jax.ShapeDtypeStruct((b, s, _TOP_K), jnp.float32),
            jax.ShapeDtypeStruct((b, s, _TOP_K), jnp.int32),
        ],
        compiler_params=pltpu.CompilerParams(
            dimension_semantics=("parallel",)),
    )(x, W_gate)
    return probs, idx


# speedup vs baseline: 1.2684x; 1.2684x over previous
import jax
import jax.numpy as jnp
from jax.experimental import pallas as pl
from jax.experimental.pallas import tpu as pltpu

_TOP_K = 8
_RC = 128


def _topk_softmax_chunk_t(st, iota, ef):
    vals = []
    idxs = []
    for k in range(_TOP_K):
        m = jnp.max(st, axis=0, keepdims=True)
        eq = st == m
        hit = jnp.where(eq, iota, ef)
        idx = jnp.min(hit, axis=0, keepdims=True)
        vals.append(m)
        idxs.append(idx)
        if k + 1 < _TOP_K:
            st = jnp.where(eq, -jnp.inf, st)
    v = jnp.concatenate(vals, axis=0)
    ix = jnp.concatenate(idxs, axis=0)
    ex = jnp.exp(v - v[0:1, :])
    p = ex / jnp.sum(ex, axis=0, keepdims=True)
    return p.T, ix.T.astype(jnp.int32)


def _router_body(x_ref, w_ref, probs_ref, idx_ref):
    bt = x_ref.shape[1]
    e = w_ref.shape[0]
    w = w_ref[...]
    iota = jax.lax.broadcasted_iota(jnp.int32, (e, _RC), 0).astype(jnp.float32)
    ef = float(e)
    n = bt // _RC

    def dot_t(c):
        lo = c * _RC
        return jax.lax.dot_general(
            w, x_ref[0, lo:lo + _RC, :], (((1,), (1,)), ((), ())),
            preferred_element_type=jnp.float32)

    st_next = dot_t(0)
    for c in range(n):
        st = st_next
        if c + 1 < n:
            st_next = dot_t(c + 1)
        p, ix = _topk_softmax_chunk_t(st, iota, ef)
        probs_ref[0, c * _RC:(c + 1) * _RC, :] = p
        idx_ref[0, c * _RC:(c + 1) * _RC, :] = ix


def kernel(x, W_gate):
    b, s, d = x.shape
    e = W_gate.shape[0]
    t = b * s
    bt = min(1024, s)
    grid = (t // bt,)
    spb = s // bt
    probs, idx = pl.pallas_call(
        _router_body,
        grid=grid,
        in_specs=[
            pl.BlockSpec((1, bt, d), lambda i: (i // spb, i % spb, 0)),
            pl.BlockSpec((e, d), lambda i: (0, 0)),
        ],
        out_specs=[
            pl.BlockSpec((1, bt, _TOP_K), lambda i: (i // spb, i % spb, 0)),
            pl.BlockSpec((1, bt, _TOP_K), lambda i: (i // spb, i % spb, 0)),
        ],
        out_shape=[
            jax.ShapeDtypeStruct((b, s, _TOP_K), jnp.float32),
            jax.ShapeDtypeStruct((b, s, _TOP_K), jnp.int32),
        ],
        compiler_params=pltpu.CompilerParams(
            dimension_semantics=("parallel",)),
    )(x, W_gate)
    return probs, idx
